# baseline (speedup 1.0000x reference)
import jax
import jax.numpy as jnp
from jax.experimental import pallas as pl
from jax.experimental.pallas import tpu as pltpu

P = 22
RADIUS = 2.0
L = 128
NSTEPS = 10
G = 4


def _ln(x, g, b):
    m = jnp.mean(x, axis=-1, keepdims=True)
    d = x - m
    v = jnp.mean(d * d, axis=-1, keepdims=True)
    return d * jax.lax.rsqrt(v + 1e-5) * g + b


def _dot(a, b):
    return jnp.dot(a, b, preferred_element_type=jnp.float32)


def _body(pos_ref, w, out_ref):
    w = jax.tree.map(lambda r: r[...], w)
    R = P * G
    E2 = P * P * G
    pos = pos_ref[:]
    mr = pos[:, 15:18]
    nf = pos[:, 3:18] - pos[:, 0:15]

    h = jnp.maximum(_dot(nf, w["enW0"]) + w["enb0"], 0.0)
    h = jnp.maximum(_dot(h, w["enW1"]) + w["enb1"], 0.0)
    x = _dot(h, w["enW2"]) + w["enb2"]
    x = _ln(x, w["enlg"], w["enlb"])

    disp = mr.reshape(P, 1, G, 3) - mr.reshape(1, P, G, 3)
    dist = jnp.sqrt(jnp.sum(disp * disp, axis=-1, keepdims=True))
    ef = jnp.concatenate([disp, dist], axis=-1).reshape(E2, 4)
    mask = (dist <= RADIUS).astype(jnp.float32).reshape(E2, 1)

    h = jnp.maximum(_dot(ef, w["eeW0"]) + w["eeb0"], 0.0)
    h = jnp.maximum(_dot(h, w["eeW1"]) + w["eeb1"], 0.0)
    e = _dot(h, w["eeW2"]) + w["eeb2"]
    e = _ln(e, w["eelg"], w["eelb"])

    for s in range(NSTEPS):
        xr = _dot(x, w["pW1r"][s])
        xs = _dot(x, w["pW1s"][s])
        t = (_dot(e, w["pW1e"][s]) + w["pb1"][s]).reshape(P, P, G, L)
        h1 = jnp.maximum(
            t + xs.reshape(P, 1, G, L) + xr.reshape(1, P, G, L), 0.0
        ).reshape(E2, L)
        h2 = jnp.maximum(_dot(h1, w["pW2"][s]) + w["pb2"][s], 0.0)
        en = _dot(h2, w["pW3"][s]) + w["pb3"][s]
        en = _ln(en, w["pelg"][s], w["pelb"][s])
        agg = jnp.sum((en * mask).reshape(P, R, L), axis=0)
        h = jnp.maximum(_dot(x, w["nW1x"][s]) + _dot(agg, w["nW1a"][s]) + w["nb1"][s], 0.0)
        h = jnp.maximum(_dot(h, w["nW2"][s]) + w["nb2"][s], 0.0)
        nn = _dot(h, w["nW3"][s]) + w["nb3"][s]
        nn = _ln(nn, w["nlg"][s], w["nlb"][s])
        x = x + nn
        e = e + en

    h = jnp.maximum(_dot(x, w["dW0"]) + w["db0"], 0.0)
    h = jnp.maximum(_dot(h, w["dW1"]) + w["db1"], 0.0)
    vel = _dot(h, w["dW2"]) + w["db2"]
    out_ref[:] = mr + vel


def kernel(current_positions, params):
    cp = current_positions
    N = cp.shape[0]
    B = N // P
    NB = B // G
    R = P * G

    posf = cp.reshape(N, 18)
    pos_t = posf.reshape(NB, G, P, 18).transpose(0, 2, 1, 3).reshape(NB * R, 18)

    def stk(f):
        return jnp.stack([f(sp) for sp in params["proc"]])

    enc_n = params["enc_node_mlp"]
    enc_e = params["enc_edge_mlp"]
    dec = params["dec_mlp"]
    w = {
        "enW0": enc_n[0]["W"], "enb0": enc_n[0]["b"][None],
        "enW1": enc_n[1]["W"], "enb1": enc_n[1]["b"][None],
        "enW2": enc_n[2]["W"], "enb2": enc_n[2]["b"][None],
        "enlg": params["enc_node_ln"]["g"][None], "enlb": params["enc_node_ln"]["b"][None],
        "eeW0": enc_e[0]["W"], "eeb0": enc_e[0]["b"][None],
        "eeW1": enc_e[1]["W"], "eeb1": enc_e[1]["b"][None],
        "eeW2": enc_e[2]["W"], "eeb2": enc_e[2]["b"][None],
        "eelg": params["enc_edge_ln"]["g"][None], "eelb": params["enc_edge_ln"]["b"][None],
        "pW1r": stk(lambda sp: sp["edge_mlp"][0]["W"][:L]),
        "pW1s": stk(lambda sp: sp["edge_mlp"][0]["W"][L:2 * L]),
        "pW1e": stk(lambda sp: sp["edge_mlp"][0]["W"][2 * L:]),
        "pb1": stk(lambda sp: sp["edge_mlp"][0]["b"][None]),
        "pW2": stk(lambda sp: sp["edge_mlp"][1]["W"]),
        "pb2": stk(lambda sp: sp["edge_mlp"][1]["b"][None]),
        "pW3": stk(lambda sp: sp["edge_mlp"][2]["W"]),
        "pb3": stk(lambda sp: sp["edge_mlp"][2]["b"][None]),
        "pelg": stk(lambda sp: sp["edge_ln"]["g"][None]),
        "pelb": stk(lambda sp: sp["edge_ln"]["b"][None]),
        "nW1x": stk(lambda sp: sp["node_mlp"][0]["W"][:L]),
        "nW1a": stk(lambda sp: sp["node_mlp"][0]["W"][L:]),
        "nb1": stk(lambda sp: sp["node_mlp"][0]["b"][None]),
        "nW2": stk(lambda sp: sp["node_mlp"][1]["W"]),
        "nb2": stk(lambda sp: sp["node_mlp"][1]["b"][None]),
        "nW3": stk(lambda sp: sp["node_mlp"][2]["W"]),
        "nb3": stk(lambda sp: sp["node_mlp"][2]["b"][None]),
        "nlg": stk(lambda sp: sp["node_ln"]["g"][None]),
        "nlb": stk(lambda sp: sp["node_ln"]["b"][None]),
        "dW0": dec[0]["W"], "db0": dec[0]["b"][None],
        "dW1": dec[1]["W"], "db1": dec[1]["b"][None],
        "dW2": dec[2]["W"], "db2": dec[2]["b"][None],
    }

    pos_spec = pl.BlockSpec((R, 18), lambda g: (g, 0))
    w_specs = jax.tree.map(
        lambda a: pl.BlockSpec(a.shape, lambda g, nd=a.ndim: (0,) * nd), w
    )

    out_t = pl.pallas_call(
        _body,
        grid=(NB,),
        in_specs=(pos_spec, w_specs),
        out_specs=pl.BlockSpec((R, 3), lambda g: (g, 0)),
        out_shape=jax.ShapeDtypeStruct((NB * R, 3), jnp.float32),
        compiler_params=pltpu.CompilerParams(
            dimension_semantics=("parallel",),
        ),
    )(pos_t, w)

    out = out_t.reshape(NB, P, G, 3).transpose(0, 2, 1, 3).reshape(N, 3)
    return out


# speedup vs baseline: 1.1002x; 1.1002x over previous
import numpy as np
import jax
import jax.numpy as jnp
from jax.experimental import pallas as pl
from jax.experimental.pallas import tpu as pltpu

P = 22
RADIUS = 2.0
L = 128
NSTEPS = 10
G = 8


def _ln(x, g, b, J):
    m = _dot(x, J)
    d = x - m
    v = _dot(d * d, J)
    return d * jax.lax.rsqrt(v + 1e-5) * g + b


def _dot(a, b):
    return jnp.dot(a, b, preferred_element_type=jnp.float32)


def _body(pos_ref, w, out_ref):
    w = jax.tree.map(lambda r: r[...], w)
    R = P * G
    E2 = P * P * G
    pos = pos_ref[:]
    mr = pos[:, 15:18]
    nf = pos[:, 3:18] - pos[:, 0:15]

    J = w["J"]
    h = jnp.maximum(_dot(nf, w["enW0"]) + w["enb0"], 0.0)
    h = jnp.maximum(_dot(h, w["enW1"]) + w["enb1"], 0.0)
    x = _dot(h, w["enW2"]) + w["enb2"]
    x = _ln(x, w["enlg"], w["enlb"], J)

    disp = mr.reshape(P, 1, G, 3) - mr.reshape(1, P, G, 3)
    dist = jnp.sqrt(jnp.sum(disp * disp, axis=-1, keepdims=True))
    ef = jnp.concatenate([disp, dist], axis=-1).reshape(E2, 4)
    mask = (dist <= RADIUS).astype(jnp.float32).reshape(E2, 1)

    h = jnp.maximum(_dot(ef, w["eeW0"]) + w["eeb0"], 0.0)
    h = jnp.maximum(_dot(h, w["eeW1"]) + w["eeb1"], 0.0)
    e = _dot(h, w["eeW2"]) + w["eeb2"]
    e = _ln(e, w["eelg"], w["eelb"], J)

    A = w["A"]
    for s in range(NSTEPS):
        xs = _dot(x, w["pW1s"][s])
        xr = _dot(x, w["pW1r"][s])
        cat = jnp.concatenate([xs, xr], axis=0)
        h1 = jnp.maximum(
            _dot(e, w["pW1e"][s]) + _dot(A, cat) + w["pb1"][s], 0.0
        )
        h2 = jnp.maximum(_dot(h1, w["pW2"][s]) + w["pb2"][s], 0.0)
        en = _dot(h2, w["pW3"][s]) + w["pb3"][s]
        en = _ln(en, w["pelg"][s], w["pelb"][s], J)
        agg = jnp.sum((en * mask).reshape(P, R, L), axis=0)
        h = jnp.maximum(_dot(x, w["nW1x"][s]) + _dot(agg, w["nW1a"][s]) + w["nb1"][s], 0.0)
        h = jnp.maximum(_dot(h, w["nW2"][s]) + w["nb2"][s], 0.0)
        nn = _dot(h, w["nW3"][s]) + w["nb3"][s]
        nn = _ln(nn, w["nlg"][s], w["nlb"][s], J)
        x = x + nn
        e = e + en

    h = jnp.maximum(_dot(x, w["dW0"]) + w["db0"], 0.0)
    h = jnp.maximum(_dot(h, w["dW1"]) + w["db1"], 0.0)
    vel = _dot(h, w["dW2"]) + w["db2"]
    out_ref[:] = mr + vel


def kernel(current_positions, params):
    cp = current_positions
    N = cp.shape[0]
    B = N // P
    NB = B // G
    R = P * G

    posf = cp.reshape(N, 18)
    pos_t = posf.reshape(NB, G, P, 18).transpose(0, 2, 1, 3).reshape(NB * R, 18)

    E2 = P * P * G
    rr = np.arange(E2)
    ii = rr // (P * G)
    jj = (rr // G) % P
    bb = rr % G
    A_np = np.zeros((E2, 2 * R), np.float32)
    A_np[rr, ii * G + bb] = 1.0
    A_np[rr, R + jj * G + bb] = 1.0

    def stk(f):
        return jnp.stack([f(sp) for sp in params["proc"]])

    enc_n = params["enc_node_mlp"]
    enc_e = params["enc_edge_mlp"]
    dec = params["dec_mlp"]
    w = {
        "enW0": enc_n[0]["W"], "enb0": enc_n[0]["b"][None],
        "enW1": enc_n[1]["W"], "enb1": enc_n[1]["b"][None],
        "enW2": enc_n[2]["W"], "enb2": enc_n[2]["b"][None],
        "enlg": params["enc_node_ln"]["g"][None], "enlb": params["enc_node_ln"]["b"][None],
        "eeW0": enc_e[0]["W"], "eeb0": enc_e[0]["b"][None],
        "eeW1": enc_e[1]["W"], "eeb1": enc_e[1]["b"][None],
        "eeW2": enc_e[2]["W"], "eeb2": enc_e[2]["b"][None],
        "eelg": params["enc_edge_ln"]["g"][None], "eelb": params["enc_edge_ln"]["b"][None],
        "pW1r": stk(lambda sp: sp["edge_mlp"][0]["W"][:L]),
        "pW1s": stk(lambda sp: sp["edge_mlp"][0]["W"][L:2 * L]),
        "pW1e": stk(lambda sp: sp["edge_mlp"][0]["W"][2 * L:]),
        "pb1": stk(lambda sp: sp["edge_mlp"][0]["b"][None]),
        "pW2": stk(lambda sp: sp["edge_mlp"][1]["W"]),
        "pb2": stk(lambda sp: sp["edge_mlp"][1]["b"][None]),
        "pW3": stk(lambda sp: sp["edge_mlp"][2]["W"]),
        "pb3": stk(lambda sp: sp["edge_mlp"][2]["b"][None]),
        "pelg": stk(lambda sp: sp["edge_ln"]["g"][None]),
        "pelb": stk(lambda sp: sp["edge_ln"]["b"][None]),
        "nW1x": stk(lambda sp: sp["node_mlp"][0]["W"][:L]),
        "nW1a": stk(lambda sp: sp["node_mlp"][0]["W"][L:]),
        "nb1": stk(lambda sp: sp["node_mlp"][0]["b"][None]),
        "nW2": stk(lambda sp: sp["node_mlp"][1]["W"]),
        "nb2": stk(lambda sp: sp["node_mlp"][1]["b"][None]),
        "nW3": stk(lambda sp: sp["node_mlp"][2]["W"]),
        "nb3": stk(lambda sp: sp["node_mlp"][2]["b"][None]),
        "nlg": stk(lambda sp: sp["node_ln"]["g"][None]),
        "nlb": stk(lambda sp: sp["node_ln"]["b"][None]),
        "J": jnp.full((L, L), 1.0 / L, jnp.float32),
        "A": jnp.asarray(A_np),
        "dW0": dec[0]["W"], "db0": dec[0]["b"][None],
        "dW1": dec[1]["W"], "db1": dec[1]["b"][None],
        "dW2": dec[2]["W"], "db2": dec[2]["b"][None],
    }

    pos_spec = pl.BlockSpec((R, 18), lambda g: (g, 0))
    w_specs = jax.tree.map(
        lambda a: pl.BlockSpec(a.shape, lambda g, nd=a.ndim: (0,) * nd), w
    )

    out_t = pl.pallas_call(
        _body,
        grid=(NB,),
        in_specs=(pos_spec, w_specs),
        out_specs=pl.BlockSpec((R, 3), lambda g: (g, 0)),
        out_shape=jax.ShapeDtypeStruct((NB * R, 3), jnp.float32),
        compiler_params=pltpu.CompilerParams(
            dimension_semantics=("parallel",),
        ),
    )(pos_t, w)

    out = out_t.reshape(NB, P, G, 3).transpose(0, 2, 1, 3).reshape(N, 3)
    return out


# speedup vs baseline: 1.3428x; 1.2205x over previous
import numpy as np
import jax
import jax.numpy as jnp
from jax.experimental import pallas as pl
from jax.experimental.pallas import tpu as pltpu

P = 22
RADIUS = 2.0
L = 128
NSTEPS = 10
G = 8


def _ln(x, g, b, J):
    m = _dot(x, J)
    d = x - m
    v = _dot(d * d, J)
    return d * jax.lax.rsqrt(v + 1e-5) * g + b


def _dot(a, b):
    return jnp.dot(a, b, preferred_element_type=jnp.float32)


def _body(pos_ref, w, out_ref):
    w = jax.tree.map(lambda r: r[...], w)
    R = P * G
    E2 = P * P * G
    pos = pos_ref[:]
    mr = pos[:, 15:18]
    nf = pos[:, 3:18] - pos[:, 0:15]

    J = w["J"]
    h = jnp.maximum(_dot(nf, w["enW0"]) + w["enb0"], 0.0)
    h = jnp.maximum(_dot(h, w["enW1"]) + w["enb1"], 0.0)
    x = _dot(h, w["enW2"]) + w["enb2"]
    x = _ln(x, w["enlg"], w["enlb"], J)

    disp = mr.reshape(P, 1, G, 3) - mr.reshape(1, P, G, 3)
    dist = jnp.sqrt(jnp.sum(disp * disp, axis=-1, keepdims=True))
    ef = jnp.concatenate([disp, dist], axis=-1).reshape(E2, 4)
    mask = (dist <= RADIUS).astype(jnp.float32).reshape(E2, 1)

    h = jnp.maximum(_dot(ef, w["eeW0"]) + w["eeb0"], 0.0)
    h = jnp.maximum(_dot(h, w["eeW1"]) + w["eeb1"], 0.0)
    e = _dot(h, w["eeW2"]) + w["eeb2"]
    e = _ln(e, w["eelg"], w["eelb"], J)

    for s in range(NSTEPS):
        xs = _dot(x, w["pW1s"][s])
        xr = _dot(x, w["pW1r"][s])
        t = (_dot(e, w["pW1e"][s]) + w["pb1"][s]).reshape(P, P, G, L)
        h1 = jnp.maximum(
            t + xs.reshape(P, 1, G, L) + xr.reshape(1, P, G, L), 0.0
        ).reshape(E2, L)
        h2 = jnp.maximum(_dot(h1, w["pW2"][s]) + w["pb2"][s], 0.0)
        en = _dot(h2, w["pW3"][s]) + w["pb3"][s]
        en = _ln(en, w["pelg"][s], w["pelb"][s], J)
        agg = jnp.sum((en * mask).reshape(P, R, L), axis=0)
        h = jnp.maximum(_dot(x, w["nW1x"][s]) + _dot(agg, w["nW1a"][s]) + w["nb1"][s], 0.0)
        h = jnp.maximum(_dot(h, w["nW2"][s]) + w["nb2"][s], 0.0)
        nn = _dot(h, w["nW3"][s]) + w["nb3"][s]
        nn = _ln(nn, w["nlg"][s], w["nlb"][s], J)
        x = x + nn
        e = e + en

    h = jnp.maximum(_dot(x, w["dW0"]) + w["db0"], 0.0)
    h = jnp.maximum(_dot(h, w["dW1"]) + w["db1"], 0.0)
    vel = _dot(h, w["dW2"]) + w["db2"]
    out_ref[:] = mr + vel


def kernel(current_positions, params):
    cp = current_positions
    N = cp.shape[0]
    B = N // P
    NB = B // G
    R = P * G

    posf = cp.reshape(N, 18)
    pos_t = posf.reshape(NB, G, P, 18).transpose(0, 2, 1, 3).reshape(NB * R, 18)

    def stk(f):
        return jnp.stack([f(sp) for sp in params["proc"]])

    enc_n = params["enc_node_mlp"]
    enc_e = params["enc_edge_mlp"]
    dec = params["dec_mlp"]
    w = {
        "enW0": enc_n[0]["W"], "enb0": enc_n[0]["b"][None],
        "enW1": enc_n[1]["W"], "enb1": enc_n[1]["b"][None],
        "enW2": enc_n[2]["W"], "enb2": enc_n[2]["b"][None],
        "enlg": params["enc_node_ln"]["g"][None], "enlb": params["enc_node_ln"]["b"][None],
        "eeW0": enc_e[0]["W"], "eeb0": enc_e[0]["b"][None],
        "eeW1": enc_e[1]["W"], "eeb1": enc_e[1]["b"][None],
        "eeW2": enc_e[2]["W"], "eeb2": enc_e[2]["b"][None],
        "eelg": params["enc_edge_ln"]["g"][None], "eelb": params["enc_edge_ln"]["b"][None],
        "pW1r": stk(lambda sp: sp["edge_mlp"][0]["W"][:L]),
        "pW1s": stk(lambda sp: sp["edge_mlp"][0]["W"][L:2 * L]),
        "pW1e": stk(lambda sp: sp["edge_mlp"][0]["W"][2 * L:]),
        "pb1": stk(lambda sp: sp["edge_mlp"][0]["b"][None]),
        "pW2": stk(lambda sp: sp["edge_mlp"][1]["W"]),
        "pb2": stk(lambda sp: sp["edge_mlp"][1]["b"][None]),
        "pW3": stk(lambda sp: sp["edge_mlp"][2]["W"]),
        "pb3": stk(lambda sp: sp["edge_mlp"][2]["b"][None]),
        "pelg": stk(lambda sp: sp["edge_ln"]["g"][None]),
        "pelb": stk(lambda sp: sp["edge_ln"]["b"][None]),
        "nW1x": stk(lambda sp: sp["node_mlp"][0]["W"][:L]),
        "nW1a": stk(lambda sp: sp["node_mlp"][0]["W"][L:]),
        "nb1": stk(lambda sp: sp["node_mlp"][0]["b"][None]),
        "nW2": stk(lambda sp: sp["node_mlp"][1]["W"]),
        "nb2": stk(lambda sp: sp["node_mlp"][1]["b"][None]),
        "nW3": stk(lambda sp: sp["node_mlp"][2]["W"]),
        "nb3": stk(lambda sp: sp["node_mlp"][2]["b"][None]),
        "nlg": stk(lambda sp: sp["node_ln"]["g"][None]),
        "nlb": stk(lambda sp: sp["node_ln"]["b"][None]),
        "J": jnp.full((L, L), 1.0 / L, jnp.float32),
        "dW0": dec[0]["W"], "db0": dec[0]["b"][None],
        "dW1": dec[1]["W"], "db1": dec[1]["b"][None],
        "dW2": dec[2]["W"], "db2": dec[2]["b"][None],
    }

    pos_spec = pl.BlockSpec((R, 18), lambda g: (g, 0))
    w_specs = jax.tree.map(
        lambda a: pl.BlockSpec(a.shape, lambda g, nd=a.ndim: (0,) * nd), w
    )

    out_t = pl.pallas_call(
        _body,
        grid=(NB,),
        in_specs=(pos_spec, w_specs),
        out_specs=pl.BlockSpec((R, 3), lambda g: (g, 0)),
        out_shape=jax.ShapeDtypeStruct((NB * R, 3), jnp.float32),
        compiler_params=pltpu.CompilerParams(
            dimension_semantics=("parallel",),
        ),
    )(pos_t, w)

    out = out_t.reshape(NB, P, G, 3).transpose(0, 2, 1, 3).reshape(N, 3)
    return out
